# Initial kernel scaffold; baseline (speedup 1.0000x reference)
import jax
import jax.numpy as jnp
from jax.experimental import pallas as pl


def kernel(x, table):
    raise NotImplementedError("write your pallas kernel here")


# speedup vs baseline: 4.2625x; 4.2625x over previous
import functools

import jax
import jax.numpy as jnp
from jax import lax
from jax.experimental import pallas as pl
from jax.experimental.pallas import tpu as pltpu
from jax.experimental.pallas import tpu_sc as plsc

NC = 2
NS = 16
NW = NC * NS

VOCAB = 100000
D = 64
CHUNK = 128
NBUF = 8
LAG = 4


def _make_sc_gather(n_chunks: int):
    mesh = plsc.VectorSubcoreMesh(core_axis_name="c", subcore_axis_name="s")

    @functools.partial(
        pl.kernel,
        mesh=mesh,
        out_type=jax.ShapeDtypeStruct((NW, n_chunks, CHUNK, D), jnp.float32),
        compiler_params=pltpu.CompilerParams(use_tc_tiling_on_sc=False),
        scratch_types=[
            pltpu.VMEM((n_chunks, CHUNK), jnp.int32),
            pltpu.VMEM((NBUF, CHUNK, D), jnp.float32),
            pltpu.SemaphoreType.DMA((NBUF,)),
            pltpu.SemaphoreType.DMA((NBUF,)),
        ],
    )
    def k(table_hbm, idx_hbm, out_hbm, idx_v, rows_v, sem_g, sem_o):
        wid = lax.axis_index("s") * NC + lax.axis_index("c")

        pltpu.sync_copy(idx_hbm.at[wid], idx_v)

        def start_gather(j, b):
            pltpu.async_copy(table_hbm.at[idx_v.at[j]], rows_v.at[b],
                             sem_g.at[b])

        def wait_gather(j, b):
            pltpu.make_async_copy(table_hbm.at[idx_v.at[j]], rows_v.at[b],
                                  sem_g.at[b]).wait()

        def start_out(i, b):
            pltpu.async_copy(rows_v.at[b], out_hbm.at[wid, i], sem_o.at[b])

        def wait_out(i, b):
            pltpu.make_async_copy(rows_v.at[b], out_hbm.at[wid, i],
                                  sem_o.at[b]).wait()

        for j in range(NBUF):
            start_gather(j, j)
            if j >= LAG:
                i = j - LAG
                wait_gather(i, i % NBUF)
                start_out(i, i % NBUF)

        def body(it, _):
            base = it * NBUF
            for b in range(NBUF):
                j = base + b
                wait_out(j - NBUF, b)
                start_gather(j, b)
                bi = (b - LAG) % NBUF
                i = j - LAG
                wait_gather(i, bi)
                start_out(i, bi)
            return _

        lax.fori_loop(1, n_chunks // NBUF, body, 0, unroll=False)

        for i in range(n_chunks - LAG, n_chunks):
            bi = i % NBUF
            wait_gather(i, bi)
            start_out(i, bi)
        for b in range(NBUF):
            wait_out(n_chunks - NBUF + b, b)

    return k


@jax.jit
def kernel(x, table):
    batch, hist = x.shape
    total = batch * hist
    n_chunks = total // (NW * CHUNK)
    idx = x.reshape(NW, n_chunks, CHUNK).astype(jnp.int32)
    out = _make_sc_gather(n_chunks)(table, idx)
    return out.reshape(batch, hist, D)
